# baseline (speedup 1.0000x reference)
import functools

import jax
import jax.numpy as jnp
from jax import lax
from jax.experimental import pallas as pl
from jax.experimental.pallas import tpu as pltpu
from jax.experimental.pallas import tpu_sc as plsc

N = 10000
D = 128
DH = D // 2
W_H = 80
C = 128
NC, NS = 2, 16
ROWS_PER_TILE = N // NS


def _tc1_body(x_ref, wa_ref, wb_ref, be1_ref, p_ref, q_ref):
    xb = x_ref[...]
    p_ref[...] = jnp.dot(xb, wa_ref[0], preferred_element_type=jnp.float32) + be1_ref[0]
    q_ref[...] = jnp.dot(xb, wb_ref[0], preferred_element_type=jnp.float32)


def _tc1(x, w_a, w_b, b_e1):
    r = 1000
    grid = (N // r, NC)
    return pl.pallas_call(
        _tc1_body,
        grid=grid,
        in_specs=[
            pl.BlockSpec((r, D), lambda i, j: (i, 0)),
            pl.BlockSpec((1, D, DH), lambda i, j: (j, 0, 0)),
            pl.BlockSpec((1, D, DH), lambda i, j: (j, 0, 0)),
            pl.BlockSpec((1, 1, DH), lambda i, j: (j, 0, 0)),
        ],
        out_specs=[
            pl.BlockSpec((r, DH), lambda i, j: (i + (N // r) * j, 0)),
            pl.BlockSpec((r, DH), lambda i, j: (i + (N // r) * j, 0)),
        ],
        out_shape=[
            jax.ShapeDtypeStruct((NC * N, DH), jnp.float32),
            jax.ShapeDtypeStruct((NC * N, DH), jnp.float32),
        ],
    )(x, w_a, w_b, b_e1)


N_ACC = N + 16
RPT = N_ACC // NS


def _make_sc_edge(n_edges):
    mesh = plsc.VectorSubcoreMesh(core_axis_name="c", subcore_axis_name="s")
    per_sub = -(-n_edges // NS)
    n_chunks = -(-per_sub // C)
    n_chunks += (-n_chunks) % 4
    kmax = n_chunks + 2

    @functools.partial(
        pl.kernel,
        out_type=jax.ShapeDtypeStruct((NC, N_ACC, W_H), jnp.float32),
        mesh=mesh,
        compiler_params=pltpu.CompilerParams(use_tc_tiling_on_sc=False,
                                             needs_layout_passes=False),
        scratch_types=[
            [pltpu.VMEM((C,), jnp.int32)] * 4,
            [pltpu.VMEM((C,), jnp.int32)] * 4,
            [pltpu.VMEM((C,), jnp.int32)] * 2,
            [pltpu.VMEM((C,), jnp.int32)] * 2,
            [[pltpu.VMEM((C,), jnp.int32)] * 6] * 2,
            [pltpu.VMEM((C, DH), jnp.float32)] * 2,
            [pltpu.VMEM((C, DH), jnp.float32)] * 2,
            [[pltpu.VMEM((C,), jnp.float32)] * 6] * 2,
            [pltpu.VMEM((C, W_H), jnp.float32)] * 2,
            pltpu.VMEM((DH,), jnp.float32),
            pltpu.VMEM_SHARED((N_ACC, W_H), jnp.float32),
            [pltpu.SemaphoreType.DMA] * 2,
            [pltpu.SemaphoreType.DMA] * 2,
        ],
    )
    def sc_edge(p_hbm, q_hbm, pos_hbm, rowg_hbm, colg_hbm, wd_hbm, out_hbm,
                rowix, colix, rofs, cofs, pidx, pi, qj, pcomp, hb, wd, hagg,
                semi, semg):
        cid = lax.axis_index("c")
        sid = lax.axis_index("s")
        r0 = sid * RPT

        pltpu.sync_copy(wd_hbm.at[cid], wd)

        zv = jnp.zeros((16,), jnp.float32)
        ones0 = jnp.where(lax.iota(jnp.int32, 16) == 0,
                          jnp.full((16,), 1.0, jnp.float32), zv)

        def zero_hbuf(e, carry):
            for j in range(W_H // 16):
                hb[0][e, pl.ds(16 * j, 16)] = zv
                hb[1][e, pl.ds(16 * j, 16)] = zv
            return carry

        lax.fori_loop(0, C, zero_hbuf, 0)

        for t in range(4):
            pltpu.sync_copy(hb[0], hagg.at[pl.ds(r0 + t * C, C)])
        pltpu.sync_copy(hb[0].at[pl.ds(0, RPT - 4 * C)],
                        hagg.at[pl.ds(r0 + 4 * C, RPT - 4 * C)])

        def set_deg_col(e, carry):
            hb[0][e, pl.ds(DH, 16)] = ones0
            hb[1][e, pl.ds(DH, 16)] = ones0
            return carry

        lax.fori_loop(0, C, set_deg_col, 0)
        plsc.subcore_barrier()

        tb = cid * N
        nm1 = jnp.full((16,), N - 1, jnp.int32)
        one_i = jnp.full((16,), 1, jnp.int32)
        two_i = jnp.full((16,), 2, jnp.int32)
        wds = [wd[pl.ds(16 * j, 16)] for j in range(DH // 16)]

        def issue_idxcopy(k, b4):
            s = semi[b4 & 1]
            pltpu.async_copy(rowg_hbm.at[sid, k], rowix[b4], s)
            pltpu.async_copy(colg_hbm.at[sid, k], colix[b4], s)

        def drain_idxcopy(k, b4):
            s = semi[b4 & 1]
            pltpu.make_async_copy(rowg_hbm.at[sid, k], rowix[b4], s).wait()
            pltpu.make_async_copy(colg_hbm.at[sid, k], colix[b4], s).wait()

        def idx_compute(b4, b2):
            for g in range(C // 16):
                sl = pl.ds(g * 16, 16)
                rvc = jnp.minimum(rowix[b4][sl], nm1)
                cvc = jnp.minimum(colix[b4][sl], nm1)
                rofs[b2][sl] = rvc + tb
                cofs[b2][sl] = cvc + tb
                r3 = rvc + rvc + rvc
                c3 = cvc + cvc + cvc
                pidx[b2][0][sl] = r3
                pidx[b2][1][sl] = r3 + one_i
                pidx[b2][2][sl] = r3 + two_i
                pidx[b2][3][sl] = c3
                pidx[b2][4][sl] = c3 + one_i
                pidx[b2][5][sl] = c3 + two_i

        def issue_gathers(b2):
            pltpu.async_copy(p_hbm.at[rofs[b2]], pi[b2], semg[b2])
            pltpu.async_copy(q_hbm.at[cofs[b2]], qj[b2], semg[b2])
            for t in range(6):
                pltpu.async_copy(pos_hbm.at[pidx[b2][t]], pcomp[b2][t], semg[b2])

        def drain_gathers(b2):
            pltpu.make_async_copy(p_hbm.at[rofs[b2]], pi[b2], semg[b2]).wait()
            pltpu.make_async_copy(q_hbm.at[cofs[b2]], qj[b2], semg[b2]).wait()
            for t in range(6):
                pltpu.make_async_copy(pos_hbm.at[pidx[b2][t]], pcomp[b2][t],
                                      semg[b2]).wait()


        def compute_h(b2):
            pxi, pyi, pzi, pxj, pyj, pzj = pcomp[b2]

            def h_body(g, hcarry):
                sl16 = pl.ds(g * 16, 16)
                dx = pxi[sl16] - pxj[sl16]
                dy = pyi[sl16] - pyj[sl16]
                dz = pzi[sl16] - pzj[sl16]
                d2 = dx * dx + dy * dy + dz * dz
                bits = lax.bitcast_convert_type(d2, jnp.int32)
                y = lax.bitcast_convert_type(
                    jnp.full((16,), 0x5F3759DF, jnp.int32) - (bits >> 1),
                    jnp.float32)
                half = d2 * 0.5
                y = y * (1.5 - half * y * y)
                y = y * (1.5 - half * y * y)
                y = y * (1.5 - half * y * y)
                dv = jnp.where(d2 > 0.0, d2 * y, zv)
                for l in range(16):
                    ds = dv[l]
                    e = g * 16 + l
                    for j in range(DH // 16):
                        sl = pl.ds(16 * j, 16)
                        hb[b2][e, sl] = jnp.maximum(
                            pi[b2][e, sl] + qj[b2][e, sl] + ds * wds[j], 0.0)
                return hcarry

            lax.fori_loop(0, C // 16, h_body, 0)

        def sync_scatter(b4, b2):
            pltpu.sync_copy(hb[b2], hagg.at[rowix[b4]], add=True)

        issue_idxcopy(0, 0)
        issue_idxcopy(1, 1)
        drain_idxcopy(0, 0)
        idx_compute(0, 0)
        issue_gathers(0)
        issue_idxcopy(2, 2)

        def quad(t, carry):
            for b in (0, 1, 2, 3):
                k = 4 * t + b
                b2 = b & 1
                drain_idxcopy(k + 1, (b + 1) & 3)
                idx_compute((b + 1) & 3, b2 ^ 1)
                issue_gathers(b2 ^ 1)
                drain_gathers(b2)
                issue_idxcopy(k + 2, (b + 2) & 3)
                compute_h(b2)
                sync_scatter(b, b2)
            return carry

        lax.fori_loop(0, n_chunks // 4, quad, 0)
        drain_gathers(0)
        drain_idxcopy(n_chunks, n_chunks & 3)
        drain_idxcopy(n_chunks + 1, (n_chunks + 1) & 3)

        plsc.subcore_barrier()
        pltpu.sync_copy(hagg.at[pl.ds(r0, RPT)],
                        out_hbm.at[cid, pl.ds(r0, RPT)])

    return sc_edge, n_chunks, kmax


def _tc2_body(ha_ref, x_ref, we2a_ref, we2b_ref, be2_ref, wn1_ref, bn1_ref,
              wn2_ref, bn2_ref, g_ref, b_ref, o_ref):
    a = ha_ref[...]
    h0 = a[0, :, 0:DH]
    h1 = a[1, :, 0:DH]
    deg = a[0, :, DH:DH + 1]
    agg = (jnp.dot(h0, we2a_ref[...], preferred_element_type=jnp.float32)
           + jnp.dot(h1, we2b_ref[...], preferred_element_type=jnp.float32)
           + deg * be2_ref[...])
    xb = x_ref[...]
    w1 = wn1_ref[...]
    h2 = jnp.maximum(
        jnp.dot(xb, w1[0:D], preferred_element_type=jnp.float32)
        + jnp.dot(agg, w1[D:2 * D], preferred_element_type=jnp.float32)
        + bn1_ref[...], 0.0)
    out = jnp.dot(h2, wn2_ref[...], preferred_element_type=jnp.float32) + bn2_ref[...] + xb
    mu = jnp.mean(out, axis=-1, keepdims=True)
    cen = out - mu
    var = jnp.mean(cen * cen, axis=-1, keepdims=True)
    o_ref[...] = cen * lax.rsqrt(var + 1e-5) * g_ref[...] + b_ref[...]


def _tc2(hagg, x, w_e2, b_e2, w_n1, b_n1, w_n2, b_n2, ln_g, ln_b):
    r = 1000
    grid = (N // r,)
    return pl.pallas_call(
        _tc2_body,
        grid=grid,
        in_specs=[
            pl.BlockSpec((NC, r, W_H), lambda i: (0, i, 0)),
            pl.BlockSpec((r, D), lambda i: (i, 0)),
            pl.BlockSpec((DH, D), lambda i: (0, 0)),
            pl.BlockSpec((DH, D), lambda i: (1, 0)),
            pl.BlockSpec((1, D), lambda i: (0, 0)),
            pl.BlockSpec((2 * D, D), lambda i: (0, 0)),
            pl.BlockSpec((1, D), lambda i: (0, 0)),
            pl.BlockSpec((D, D), lambda i: (0, 0)),
            pl.BlockSpec((1, D), lambda i: (0, 0)),
            pl.BlockSpec((1, D), lambda i: (0, 0)),
            pl.BlockSpec((1, D), lambda i: (0, 0)),
        ],
        out_specs=pl.BlockSpec((r, D), lambda i: (i, 0)),
        out_shape=jax.ShapeDtypeStruct((N, D), jnp.float32),
    )(hagg, x, w_e2, w_e2, b_e2, w_n1, b_n1, w_n2, b_n2, ln_g, ln_b)


def kernel(x, pos, edge_index, W_e1, b_e1, W_e2, b_e2, W_n1, b_n1, W_n2, b_n2,
           ln_g, ln_b):
    n_edges = edge_index.shape[1]

    w_a = jnp.stack([W_e1[:D, :DH], W_e1[:D, DH:]])
    w_b = jnp.stack([W_e1[D:2 * D, :DH], W_e1[D:2 * D, DH:]])
    wd2 = W_e1[2 * D].reshape(NC, DH)
    pos_t = pos.T

    sc_fn, n_chunks, kmax = _make_sc_edge(n_edges)
    per = -(-n_edges // NS)
    ei_pad = jnp.pad(edge_index, ((0, 0), (0, NS * per - n_edges)),
                     constant_values=N).reshape(2, NS, per)
    rc = jnp.pad(ei_pad, ((0, 0), (0, 0), (0, kmax * C - per)),
                 constant_values=N)
    rowg = rc[0].reshape(NS, kmax, C)
    colg = rc[1].reshape(NS, kmax, C)
    pos_flat = pos.reshape(-1)

    p2, q2 = _tc1(x, w_a, w_b, b_e1.reshape(NC, 1, DH))
    hagg = sc_fn(p2, q2, pos_flat, rowg, colg, wd2)
    out = _tc2(hagg, x, W_e2, b_e2.reshape(1, D), W_n1, b_n1.reshape(1, D),
               W_n2, b_n2.reshape(1, D), ln_g.reshape(1, D), ln_b.reshape(1, D))
    return (out, pos)


# speedup vs baseline: 1.0583x; 1.0583x over previous
import functools

import jax
import jax.numpy as jnp
from jax import lax
from jax.experimental import pallas as pl
from jax.experimental.pallas import tpu as pltpu
from jax.experimental.pallas import tpu_sc as plsc

N = 10000
D = 128
DH = D // 2
W_H = 80
C = 128
NC, NS = 2, 16
ROWS_PER_TILE = N // NS


def _tc1_body(x_ref, wa_ref, wb_ref, be1_ref, p_ref, q_ref):
    xb = x_ref[...]
    p_ref[...] = jnp.dot(xb, wa_ref[0], preferred_element_type=jnp.float32) + be1_ref[0]
    q_ref[...] = jnp.dot(xb, wb_ref[0], preferred_element_type=jnp.float32)


def _tc1(x, w_a, w_b, b_e1):
    r = 1000
    grid = (N // r, NC)
    return pl.pallas_call(
        _tc1_body,
        grid=grid,
        in_specs=[
            pl.BlockSpec((r, D), lambda i, j: (i, 0)),
            pl.BlockSpec((1, D, DH), lambda i, j: (j, 0, 0)),
            pl.BlockSpec((1, D, DH), lambda i, j: (j, 0, 0)),
            pl.BlockSpec((1, 1, DH), lambda i, j: (j, 0, 0)),
        ],
        out_specs=[
            pl.BlockSpec((r, DH), lambda i, j: (i + (N // r) * j, 0)),
            pl.BlockSpec((r, DH), lambda i, j: (i + (N // r) * j, 0)),
        ],
        out_shape=[
            jax.ShapeDtypeStruct((NC * N, DH), jnp.float32),
            jax.ShapeDtypeStruct((NC * N, DH), jnp.float32),
        ],
    )(x, w_a, w_b, b_e1)


N_ACC = N + 16
RPT = N_ACC // NS


def _make_sc_edge(n_edges):
    mesh = plsc.VectorSubcoreMesh(core_axis_name="c", subcore_axis_name="s")
    per_sub = -(-n_edges // NS)
    n_chunks = -(-per_sub // C)
    n_chunks += (-n_chunks) % 4
    kmax = n_chunks + 2

    @functools.partial(
        pl.kernel,
        out_type=jax.ShapeDtypeStruct((NC, N_ACC, W_H), jnp.float32),
        mesh=mesh,
        compiler_params=pltpu.CompilerParams(use_tc_tiling_on_sc=False,
                                             needs_layout_passes=False),
        scratch_types=[
            [pltpu.VMEM((C,), jnp.int32)] * 4,
            [pltpu.VMEM((C,), jnp.int32)] * 4,
            [pltpu.VMEM((C,), jnp.int32)] * 2,
            [pltpu.VMEM((C,), jnp.int32)] * 2,
            [[pltpu.VMEM((C,), jnp.int32)] * 6] * 2,
            [pltpu.VMEM((C, DH), jnp.float32)] * 2,
            [pltpu.VMEM((C, DH), jnp.float32)] * 2,
            [[pltpu.VMEM((C,), jnp.float32)] * 6] * 2,
            [pltpu.VMEM((C, W_H), jnp.float32)] * 2,
            pltpu.VMEM((DH,), jnp.float32),
            pltpu.VMEM_SHARED((N_ACC, W_H), jnp.float32),
            [pltpu.SemaphoreType.DMA] * 2,
            [pltpu.SemaphoreType.DMA] * 2,
            [pltpu.SemaphoreType.DMA] * 2,
        ],
    )
    def sc_edge(p_hbm, q_hbm, pos_hbm, rowg_hbm, colg_hbm, wd_hbm, out_hbm,
                rowix, colix, rofs, cofs, pidx, pi, qj, pcomp, hb, wd, hagg,
                semi, semg, sems):
        cid = lax.axis_index("c")
        sid = lax.axis_index("s")
        r0 = sid * RPT

        pltpu.sync_copy(wd_hbm.at[cid], wd)

        zv = jnp.zeros((16,), jnp.float32)
        ones0 = jnp.where(lax.iota(jnp.int32, 16) == 0,
                          jnp.full((16,), 1.0, jnp.float32), zv)

        def zero_hbuf(e, carry):
            for j in range(W_H // 16):
                hb[0][e, pl.ds(16 * j, 16)] = zv
                hb[1][e, pl.ds(16 * j, 16)] = zv
            return carry

        lax.fori_loop(0, C, zero_hbuf, 0)

        for t in range(4):
            pltpu.sync_copy(hb[0], hagg.at[pl.ds(r0 + t * C, C)])
        pltpu.sync_copy(hb[0].at[pl.ds(0, RPT - 4 * C)],
                        hagg.at[pl.ds(r0 + 4 * C, RPT - 4 * C)])

        def set_deg_col(e, carry):
            hb[0][e, pl.ds(DH, 16)] = ones0
            hb[1][e, pl.ds(DH, 16)] = ones0
            return carry

        lax.fori_loop(0, C, set_deg_col, 0)
        plsc.subcore_barrier()

        tb = cid * N
        nm1 = jnp.full((16,), N - 1, jnp.int32)
        one_i = jnp.full((16,), 1, jnp.int32)
        two_i = jnp.full((16,), 2, jnp.int32)
        wds = [wd[pl.ds(16 * j, 16)] for j in range(DH // 16)]

        def issue_idxcopy(k, b4):
            s = semi[b4 & 1]
            pltpu.async_copy(rowg_hbm.at[sid, k], rowix[b4], s)
            pltpu.async_copy(colg_hbm.at[sid, k], colix[b4], s)

        def drain_idxcopy(k, b4):
            s = semi[b4 & 1]
            pltpu.make_async_copy(rowg_hbm.at[sid, k], rowix[b4], s).wait()
            pltpu.make_async_copy(colg_hbm.at[sid, k], colix[b4], s).wait()

        def idx_compute(b4, b2):
            for g in range(C // 16):
                sl = pl.ds(g * 16, 16)
                rvc = jnp.minimum(rowix[b4][sl], nm1)
                cvc = jnp.minimum(colix[b4][sl], nm1)
                rofs[b2][sl] = rvc + tb
                cofs[b2][sl] = cvc + tb
                r3 = rvc + rvc + rvc
                c3 = cvc + cvc + cvc
                pidx[b2][0][sl] = r3
                pidx[b2][1][sl] = r3 + one_i
                pidx[b2][2][sl] = r3 + two_i
                pidx[b2][3][sl] = c3
                pidx[b2][4][sl] = c3 + one_i
                pidx[b2][5][sl] = c3 + two_i

        def issue_gathers(b2):
            pltpu.async_copy(p_hbm.at[rofs[b2]], pi[b2], semg[b2])
            pltpu.async_copy(q_hbm.at[cofs[b2]], qj[b2], semg[b2])
            for t in range(6):
                pltpu.async_copy(pos_hbm.at[pidx[b2][t]], pcomp[b2][t], semg[b2])

        def drain_gathers(b2):
            pltpu.make_async_copy(p_hbm.at[rofs[b2]], pi[b2], semg[b2]).wait()
            pltpu.make_async_copy(q_hbm.at[cofs[b2]], qj[b2], semg[b2]).wait()
            for t in range(6):
                pltpu.make_async_copy(pos_hbm.at[pidx[b2][t]], pcomp[b2][t],
                                      semg[b2]).wait()


        def compute_h(b2):
            pxi, pyi, pzi, pxj, pyj, pzj = pcomp[b2]

            def h_body(g, hcarry):
                sl16 = pl.ds(g * 16, 16)
                dx = pxi[sl16] - pxj[sl16]
                dy = pyi[sl16] - pyj[sl16]
                dz = pzi[sl16] - pzj[sl16]
                d2 = dx * dx + dy * dy + dz * dz
                bits = lax.bitcast_convert_type(d2, jnp.int32)
                y = lax.bitcast_convert_type(
                    jnp.full((16,), 0x5F3759DF, jnp.int32) - (bits >> 1),
                    jnp.float32)
                half = d2 * 0.5
                y = y * (1.5 - half * y * y)
                y = y * (1.5 - half * y * y)
                y = y * (1.5 - half * y * y)
                dv = jnp.where(d2 > 0.0, d2 * y, zv)
                for l in range(16):
                    ds = dv[l]
                    e = g * 16 + l
                    for j in range(DH // 16):
                        sl = pl.ds(16 * j, 16)
                        hb[b2][e, sl] = jnp.maximum(
                            pi[b2][e, sl] + qj[b2][e, sl] + ds * wds[j], 0.0)
                return hcarry

            lax.fori_loop(0, C // 16, h_body, 0)

        def issue_scatter(b4, b2):
            pltpu.async_copy(hb[b2], hagg.at[rowix[b4]], sems[b2], add=True)

        def drain_scatter(b4, b2):
            pltpu.make_async_copy(hb[b2], hagg.at[rowix[b4]], sems[b2]).wait()

        issue_idxcopy(0, 0)
        issue_idxcopy(1, 1)
        drain_idxcopy(0, 0)
        idx_compute(0, 0)
        issue_gathers(0)
        issue_idxcopy(2, 2)
        trash = jnp.full((16,), N, jnp.int32)
        for g in range(C // 16):
            rowix[3][pl.ds(g * 16, 16)] = trash
        issue_scatter(3, 0)
        issue_scatter(3, 1)

        def quad(t, carry):
            for b in (0, 1, 2, 3):
                k = 4 * t + b
                b2 = b & 1
                drain_idxcopy(k + 1, (b + 1) & 3)
                idx_compute((b + 1) & 3, b2 ^ 1)
                issue_gathers(b2 ^ 1)
                drain_gathers(b2)
                drain_scatter((b + 2) & 3, b2)
                issue_idxcopy(k + 2, (b + 2) & 3)
                compute_h(b2)
                issue_scatter(b, b2)
            return carry

        lax.fori_loop(0, n_chunks // 4, quad, 0)
        drain_gathers(0)
        drain_idxcopy(n_chunks, n_chunks & 3)
        drain_idxcopy(n_chunks + 1, (n_chunks + 1) & 3)
        drain_scatter((n_chunks - 2) & 3, 0)
        drain_scatter((n_chunks - 1) & 3, 1)

        plsc.subcore_barrier()
        pltpu.sync_copy(hagg.at[pl.ds(r0, RPT)],
                        out_hbm.at[cid, pl.ds(r0, RPT)])

    return sc_edge, n_chunks, kmax


def _tc2_body(ha_ref, x_ref, we2a_ref, we2b_ref, be2_ref, wn1_ref, bn1_ref,
              wn2_ref, bn2_ref, g_ref, b_ref, o_ref):
    a = ha_ref[...]
    h0 = a[0, :, 0:DH]
    h1 = a[1, :, 0:DH]
    deg = a[0, :, DH:DH + 1]
    agg = (jnp.dot(h0, we2a_ref[...], preferred_element_type=jnp.float32)
           + jnp.dot(h1, we2b_ref[...], preferred_element_type=jnp.float32)
           + deg * be2_ref[...])
    xb = x_ref[...]
    w1 = wn1_ref[...]
    h2 = jnp.maximum(
        jnp.dot(xb, w1[0:D], preferred_element_type=jnp.float32)
        + jnp.dot(agg, w1[D:2 * D], preferred_element_type=jnp.float32)
        + bn1_ref[...], 0.0)
    out = jnp.dot(h2, wn2_ref[...], preferred_element_type=jnp.float32) + bn2_ref[...] + xb
    mu = jnp.mean(out, axis=-1, keepdims=True)
    cen = out - mu
    var = jnp.mean(cen * cen, axis=-1, keepdims=True)
    o_ref[...] = cen * lax.rsqrt(var + 1e-5) * g_ref[...] + b_ref[...]


def _tc2(hagg, x, w_e2, b_e2, w_n1, b_n1, w_n2, b_n2, ln_g, ln_b):
    r = 1000
    grid = (N // r,)
    return pl.pallas_call(
        _tc2_body,
        grid=grid,
        in_specs=[
            pl.BlockSpec((NC, r, W_H), lambda i: (0, i, 0)),
            pl.BlockSpec((r, D), lambda i: (i, 0)),
            pl.BlockSpec((DH, D), lambda i: (0, 0)),
            pl.BlockSpec((DH, D), lambda i: (1, 0)),
            pl.BlockSpec((1, D), lambda i: (0, 0)),
            pl.BlockSpec((2 * D, D), lambda i: (0, 0)),
            pl.BlockSpec((1, D), lambda i: (0, 0)),
            pl.BlockSpec((D, D), lambda i: (0, 0)),
            pl.BlockSpec((1, D), lambda i: (0, 0)),
            pl.BlockSpec((1, D), lambda i: (0, 0)),
            pl.BlockSpec((1, D), lambda i: (0, 0)),
        ],
        out_specs=pl.BlockSpec((r, D), lambda i: (i, 0)),
        out_shape=jax.ShapeDtypeStruct((N, D), jnp.float32),
    )(hagg, x, w_e2, w_e2, b_e2, w_n1, b_n1, w_n2, b_n2, ln_g, ln_b)


def kernel(x, pos, edge_index, W_e1, b_e1, W_e2, b_e2, W_n1, b_n1, W_n2, b_n2,
           ln_g, ln_b):
    n_edges = edge_index.shape[1]

    w_a = jnp.stack([W_e1[:D, :DH], W_e1[:D, DH:]])
    w_b = jnp.stack([W_e1[D:2 * D, :DH], W_e1[D:2 * D, DH:]])
    wd2 = W_e1[2 * D].reshape(NC, DH)
    pos_t = pos.T

    sc_fn, n_chunks, kmax = _make_sc_edge(n_edges)
    per = -(-n_edges // NS)
    ei_pad = jnp.pad(edge_index, ((0, 0), (0, NS * per - n_edges)),
                     constant_values=N).reshape(2, NS, per)
    rc = jnp.pad(ei_pad, ((0, 0), (0, 0), (0, kmax * C - per)),
                 constant_values=N)
    rowg = rc[0].reshape(NS, kmax, C)
    colg = rc[1].reshape(NS, kmax, C)
    pos_flat = pos.reshape(-1)

    p2, q2 = _tc1(x, w_a, w_b, b_e1.reshape(NC, 1, DH))
    hagg = sc_fn(p2, q2, pos_flat, rowg, colg, wd2)
    out = _tc2(hagg, x, W_e2, b_e2.reshape(1, D), W_n1, b_n1.reshape(1, D),
               W_n2, b_n2.reshape(1, D), ln_g.reshape(1, D), ln_b.reshape(1, D))
    return (out, pos)


# speedup vs baseline: 1.3648x; 1.2896x over previous
import functools

import jax
import jax.numpy as jnp
from jax import lax
from jax.experimental import pallas as pl
from jax.experimental.pallas import tpu as pltpu
from jax.experimental.pallas import tpu_sc as plsc

N = 10000
D = 128
DH = D // 2
W_H = 80
C = 128
NC, NS = 2, 16
ROWS_PER_TILE = N // NS


def _tc1_body(x_ref, wa_ref, wb_ref, be1_ref, p_ref, q_ref):
    xb = x_ref[...]
    p_ref[...] = jnp.dot(xb, wa_ref[0], preferred_element_type=jnp.float32) + be1_ref[0]
    q_ref[...] = jnp.dot(xb, wb_ref[0], preferred_element_type=jnp.float32)


def _tc1(x, w_a, w_b, b_e1):
    r = 1000
    grid = (N // r, NC)
    return pl.pallas_call(
        _tc1_body,
        grid=grid,
        in_specs=[
            pl.BlockSpec((r, D), lambda i, j: (i, 0)),
            pl.BlockSpec((1, D, DH), lambda i, j: (j, 0, 0)),
            pl.BlockSpec((1, D, DH), lambda i, j: (j, 0, 0)),
            pl.BlockSpec((1, 1, DH), lambda i, j: (j, 0, 0)),
        ],
        out_specs=[
            pl.BlockSpec((r, DH), lambda i, j: (i + (N // r) * j, 0)),
            pl.BlockSpec((r, DH), lambda i, j: (i + (N // r) * j, 0)),
        ],
        out_shape=[
            jax.ShapeDtypeStruct((NC * N, DH), jnp.float32),
            jax.ShapeDtypeStruct((NC * N, DH), jnp.float32),
        ],
    )(x, w_a, w_b, b_e1)


N_ACC = N + 16
RPT = N_ACC // NS


def _make_sc_edge(n_edges):
    mesh = plsc.VectorSubcoreMesh(core_axis_name="c", subcore_axis_name="s")
    per_sub = -(-n_edges // NS)
    n_chunks = -(-per_sub // C)
    n_chunks += (-n_chunks) % 4
    kmax = n_chunks + 2

    @functools.partial(
        pl.kernel,
        out_type=jax.ShapeDtypeStruct((NC, N_ACC, W_H), jnp.float32),
        mesh=mesh,
        compiler_params=pltpu.CompilerParams(use_tc_tiling_on_sc=False,
                                             needs_layout_passes=False),
        scratch_types=[
            [pltpu.VMEM((C,), jnp.int32)] * 4,
            [pltpu.VMEM((C,), jnp.int32)] * 4,
            [pltpu.VMEM((C,), jnp.int32)] * 2,
            [pltpu.VMEM((C,), jnp.int32)] * 2,
            [pltpu.VMEM((C,), jnp.int32)] * 2,
            [pltpu.VMEM((C,), jnp.int32)] * 2,
            [pltpu.VMEM((C, DH), jnp.float32)] * 2,
            [pltpu.VMEM((C, DH), jnp.float32)] * 2,
            [pltpu.VMEM((C, 16), jnp.float32)] * 2,
            [pltpu.VMEM((C, 16), jnp.float32)] * 2,
            [pltpu.VMEM((C, W_H), jnp.float32)] * 2,
            pltpu.VMEM((DH,), jnp.float32),
            pltpu.VMEM_SHARED((N_ACC, W_H), jnp.float32),
            [pltpu.SemaphoreType.DMA] * 2,
            [pltpu.SemaphoreType.DMA] * 2,
            [pltpu.SemaphoreType.DMA] * 2,
        ],
    )
    def sc_edge(p_hbm, q_hbm, pos_hbm, rowg_hbm, colg_hbm, wd_hbm, out_hbm,
                rowix, colix, rofs, cofs, prix, pcix, pi, qj, posi, posj,
                hb, wd, hagg, semi, semg, sems):
        cid = lax.axis_index("c")
        sid = lax.axis_index("s")
        r0 = sid * RPT

        pltpu.sync_copy(wd_hbm.at[cid], wd)

        zv = jnp.zeros((16,), jnp.float32)
        ones0 = jnp.where(lax.iota(jnp.int32, 16) == 0,
                          jnp.full((16,), 1.0, jnp.float32), zv)

        def zero_hbuf(e, carry):
            for j in range(W_H // 16):
                hb[0][e, pl.ds(16 * j, 16)] = zv
                hb[1][e, pl.ds(16 * j, 16)] = zv
            return carry

        lax.fori_loop(0, C, zero_hbuf, 0)

        for t in range(4):
            pltpu.sync_copy(hb[0], hagg.at[pl.ds(r0 + t * C, C)])
        pltpu.sync_copy(hb[0].at[pl.ds(0, RPT - 4 * C)],
                        hagg.at[pl.ds(r0 + 4 * C, RPT - 4 * C)])

        def set_deg_col(e, carry):
            hb[0][e, pl.ds(DH, 16)] = ones0
            hb[1][e, pl.ds(DH, 16)] = ones0
            return carry

        lax.fori_loop(0, C, set_deg_col, 0)
        plsc.subcore_barrier()

        tb = cid * N
        nm1 = jnp.full((16,), N - 1, jnp.int32)
        lane = lax.iota(jnp.int32, 16)
        col0 = jnp.zeros((16,), jnp.int32)
        col1 = jnp.full((16,), 1, jnp.int32)
        col2 = jnp.full((16,), 2, jnp.int32)
        wds = [wd[pl.ds(16 * j, 16)] for j in range(DH // 16)]

        def issue_idxcopy(k, b4):
            s = semi[b4 & 1]
            pltpu.async_copy(rowg_hbm.at[sid, k], rowix[b4], s)
            pltpu.async_copy(colg_hbm.at[sid, k], colix[b4], s)

        def drain_idxcopy(k, b4):
            s = semi[b4 & 1]
            pltpu.make_async_copy(rowg_hbm.at[sid, k], rowix[b4], s).wait()
            pltpu.make_async_copy(colg_hbm.at[sid, k], colix[b4], s).wait()

        def idx_compute(b4, b2):
            for g in range(C // 16):
                sl = pl.ds(g * 16, 16)
                rvc = jnp.minimum(rowix[b4][sl], nm1)
                cvc = jnp.minimum(colix[b4][sl], nm1)
                rofs[b2][sl] = rvc + tb
                cofs[b2][sl] = cvc + tb
                prix[b2][sl] = rvc
                pcix[b2][sl] = cvc

        def issue_gathers(b2):
            pltpu.async_copy(p_hbm.at[rofs[b2]], pi[b2], semg[b2])
            pltpu.async_copy(q_hbm.at[cofs[b2]], qj[b2], semg[b2])
            pltpu.async_copy(pos_hbm.at[prix[b2]], posi[b2], semg[b2])
            pltpu.async_copy(pos_hbm.at[pcix[b2]], posj[b2], semg[b2])

        def drain_gathers(b2):
            pltpu.make_async_copy(p_hbm.at[rofs[b2]], pi[b2], semg[b2]).wait()
            pltpu.make_async_copy(q_hbm.at[cofs[b2]], qj[b2], semg[b2]).wait()
            pltpu.make_async_copy(pos_hbm.at[prix[b2]], posi[b2], semg[b2]).wait()
            pltpu.make_async_copy(pos_hbm.at[pcix[b2]], posj[b2], semg[b2]).wait()


        def compute_h(b2):
            def h_body(g, hcarry):
                e16 = g * 16 + lane
                dx = (plsc.load_gather(posi[b2], [e16, col0])
                      - plsc.load_gather(posj[b2], [e16, col0]))
                dy = (plsc.load_gather(posi[b2], [e16, col1])
                      - plsc.load_gather(posj[b2], [e16, col1]))
                dz = (plsc.load_gather(posi[b2], [e16, col2])
                      - plsc.load_gather(posj[b2], [e16, col2]))
                d2 = dx * dx + dy * dy + dz * dz
                bits = lax.bitcast_convert_type(d2, jnp.int32)
                y = lax.bitcast_convert_type(
                    jnp.full((16,), 0x5F3759DF, jnp.int32) - (bits >> 1),
                    jnp.float32)
                half = d2 * 0.5
                y = y * (1.5 - half * y * y)
                y = y * (1.5 - half * y * y)
                y = y * (1.5 - half * y * y)
                dv = jnp.where(d2 > 0.0, d2 * y, zv)
                for l in range(16):
                    ds = dv[l]
                    e = g * 16 + l
                    for j in range(DH // 16):
                        sl = pl.ds(16 * j, 16)
                        hb[b2][e, sl] = jnp.maximum(
                            pi[b2][e, sl] + qj[b2][e, sl] + ds * wds[j], 0.0)
                return hcarry

            lax.fori_loop(0, C // 16, h_body, 0)

        def issue_scatter(b4, b2):
            pltpu.async_copy(hb[b2], hagg.at[rowix[b4]], sems[b2], add=True)

        def drain_scatter(b4, b2):
            pltpu.make_async_copy(hb[b2], hagg.at[rowix[b4]], sems[b2]).wait()

        issue_idxcopy(0, 0)
        issue_idxcopy(1, 1)
        drain_idxcopy(0, 0)
        idx_compute(0, 0)
        issue_gathers(0)
        issue_idxcopy(2, 2)
        trash = jnp.full((16,), N, jnp.int32)
        for g in range(C // 16):
            rowix[3][pl.ds(g * 16, 16)] = trash
        issue_scatter(3, 0)
        issue_scatter(3, 1)

        def quad(t, carry):
            for b in (0, 1, 2, 3):
                k = 4 * t + b
                b2 = b & 1
                drain_idxcopy(k + 1, (b + 1) & 3)
                idx_compute((b + 1) & 3, b2 ^ 1)
                issue_gathers(b2 ^ 1)
                drain_gathers(b2)
                drain_scatter((b + 2) & 3, b2)
                issue_idxcopy(k + 2, (b + 2) & 3)
                compute_h(b2)
                issue_scatter(b, b2)
            return carry

        lax.fori_loop(0, n_chunks // 4, quad, 0)
        drain_gathers(0)
        drain_idxcopy(n_chunks, n_chunks & 3)
        drain_idxcopy(n_chunks + 1, (n_chunks + 1) & 3)
        drain_scatter((n_chunks - 2) & 3, 0)
        drain_scatter((n_chunks - 1) & 3, 1)

        plsc.subcore_barrier()
        pltpu.sync_copy(hagg.at[pl.ds(r0, RPT)],
                        out_hbm.at[cid, pl.ds(r0, RPT)])

    return sc_edge, n_chunks, kmax


def _tc2_body(ha_ref, x_ref, we2a_ref, we2b_ref, be2_ref, wn1_ref, bn1_ref,
              wn2_ref, bn2_ref, g_ref, b_ref, o_ref):
    a = ha_ref[...]
    h0 = a[0, :, 0:DH]
    h1 = a[1, :, 0:DH]
    deg = a[0, :, DH:DH + 1]
    agg = (jnp.dot(h0, we2a_ref[...], preferred_element_type=jnp.float32)
           + jnp.dot(h1, we2b_ref[...], preferred_element_type=jnp.float32)
           + deg * be2_ref[...])
    xb = x_ref[...]
    w1 = wn1_ref[...]
    h2 = jnp.maximum(
        jnp.dot(xb, w1[0:D], preferred_element_type=jnp.float32)
        + jnp.dot(agg, w1[D:2 * D], preferred_element_type=jnp.float32)
        + bn1_ref[...], 0.0)
    out = jnp.dot(h2, wn2_ref[...], preferred_element_type=jnp.float32) + bn2_ref[...] + xb
    mu = jnp.mean(out, axis=-1, keepdims=True)
    cen = out - mu
    var = jnp.mean(cen * cen, axis=-1, keepdims=True)
    o_ref[...] = cen * lax.rsqrt(var + 1e-5) * g_ref[...] + b_ref[...]


def _tc2(hagg, x, w_e2, b_e2, w_n1, b_n1, w_n2, b_n2, ln_g, ln_b):
    r = 1000
    grid = (N // r,)
    return pl.pallas_call(
        _tc2_body,
        grid=grid,
        in_specs=[
            pl.BlockSpec((NC, r, W_H), lambda i: (0, i, 0)),
            pl.BlockSpec((r, D), lambda i: (i, 0)),
            pl.BlockSpec((DH, D), lambda i: (0, 0)),
            pl.BlockSpec((DH, D), lambda i: (1, 0)),
            pl.BlockSpec((1, D), lambda i: (0, 0)),
            pl.BlockSpec((2 * D, D), lambda i: (0, 0)),
            pl.BlockSpec((1, D), lambda i: (0, 0)),
            pl.BlockSpec((D, D), lambda i: (0, 0)),
            pl.BlockSpec((1, D), lambda i: (0, 0)),
            pl.BlockSpec((1, D), lambda i: (0, 0)),
            pl.BlockSpec((1, D), lambda i: (0, 0)),
        ],
        out_specs=pl.BlockSpec((r, D), lambda i: (i, 0)),
        out_shape=jax.ShapeDtypeStruct((N, D), jnp.float32),
    )(hagg, x, w_e2, w_e2, b_e2, w_n1, b_n1, w_n2, b_n2, ln_g, ln_b)


def kernel(x, pos, edge_index, W_e1, b_e1, W_e2, b_e2, W_n1, b_n1, W_n2, b_n2,
           ln_g, ln_b):
    n_edges = edge_index.shape[1]

    w_a = jnp.stack([W_e1[:D, :DH], W_e1[:D, DH:]])
    w_b = jnp.stack([W_e1[D:2 * D, :DH], W_e1[D:2 * D, DH:]])
    wd2 = W_e1[2 * D].reshape(NC, DH)
    pos_t = pos.T

    sc_fn, n_chunks, kmax = _make_sc_edge(n_edges)
    per = -(-n_edges // NS)
    ei_pad = jnp.pad(edge_index, ((0, 0), (0, NS * per - n_edges)),
                     constant_values=N).reshape(2, NS, per)
    rc = jnp.pad(ei_pad, ((0, 0), (0, 0), (0, kmax * C - per)),
                 constant_values=N)
    rowg = rc[0].reshape(NS, kmax, C)
    colg = rc[1].reshape(NS, kmax, C)
    pos16 = jnp.pad(pos, ((0, 0), (0, 16 - pos.shape[1])))

    p2, q2 = _tc1(x, w_a, w_b, b_e1.reshape(NC, 1, DH))
    hagg = sc_fn(p2, q2, pos16, rowg, colg, wd2)
    out = _tc2(hagg, x, W_e2, b_e2.reshape(1, D), W_n1, b_n1.reshape(1, D),
               W_n2, b_n2.reshape(1, D), ln_g.reshape(1, D), ln_b.reshape(1, D))
    return (out, pos)
